# baseline (speedup 1.0000x reference)
import jax
import jax.numpy as jnp
from jax.experimental import pallas as pl
from jax.experimental.pallas import tpu as pltpu

EPS_DICE = 0.0001


def _dice_block_kernel(out_ref, tgt_ref, acc_ref):
    c, h, w = out_ref.shape[1:]
    hs = 8
    cls = jax.lax.broadcasted_iota(jnp.int32, (c, hs, w), 0)

    def body(k, accs):
        a_tot, a_s2 = accs
        xk = out_ref[0, :, pl.ds(k * hs, hs), :]
        tk = tgt_ref[0, pl.ds(k * hs, hs), :]
        th = jnp.tanh(xk * 0.5)
        m = tk[None, :, :] == cls
        a_tot = a_tot + th
        a_s2 = a_s2 + jnp.where(m, th + 1.0, 0.0)
        return a_tot, a_s2

    acc_tot, acc_s2 = jax.lax.fori_loop(
        0,
        h // hs,
        body,
        (jnp.zeros((c, hs, w), jnp.float32), jnp.zeros((c, hs, w), jnp.float32)),
    )

    def cbody(k, a_cnt):
        tk = tgt_ref[0, pl.ds(k * hs, hs), :]
        return a_cnt + jnp.where(tk[None, :, :] == cls, 1.0, 0.0)

    acc_cnt = jax.lax.fori_loop(
        0, h // hs, cbody, jnp.zeros((c, hs, w), jnp.float32)
    )

    t_tot = jnp.sum(acc_tot, axis=(1, 2))
    s2 = jnp.sum(acc_s2, axis=(1, 2))
    cnt = jnp.sum(acc_cnt, axis=(1, 2))
    acc_ref[0, 0] = jnp.concatenate([t_tot, s2, cnt])


@jax.jit
def kernel(output, target):
    b, c, h, w = output.shape
    tgt = target.astype(jnp.int32)
    acc = pl.pallas_call(
        _dice_block_kernel,
        grid=(b,),
        in_specs=[
            pl.BlockSpec((1, c, h, w), lambda i: (i, 0, 0, 0)),
            pl.BlockSpec((1, h, w), lambda i: (i, 0, 0)),
        ],
        out_specs=pl.BlockSpec((1, 1, 3 * c), lambda i: (i, 0, 0)),
        out_shape=jax.ShapeDtypeStruct((b, 1, 3 * c), jnp.float32),
        compiler_params=pltpu.CompilerParams(
            dimension_semantics=("arbitrary",),
            vmem_limit_bytes=100 * 1024 * 1024,
        ),
    )(output, tgt)
    t_tot = acc[:, 0, :c]
    s2 = acc[:, 0, c : 2 * c]
    cnt = acc[:, 0, 2 * c :]
    hw = jnp.float32(h * w)
    sig_sum = 0.5 * t_tot + 0.5 * hw
    loss_per_channel = 1.0 - s2 / (sig_sum + cnt + EPS_DICE)
    return loss_per_channel.sum(axis=1) / c


# speedup vs baseline: 1.4891x; 1.4891x over previous
import jax
import jax.numpy as jnp
from jax.experimental import pallas as pl
from jax.experimental.pallas import tpu as pltpu

EPS_DICE = 0.0001


def _dice_block_kernel(out_ref, tgt_ref, loss_ref):
    c, h, w = out_ref.shape[1:]
    x = out_ref[0]
    t = tgt_ref[0]
    cls = jax.lax.broadcasted_iota(jnp.int32, x.shape, 0)
    th = jnp.tanh(x * 0.5)
    m = t[None, :, :] == cls
    t_tot = jnp.sum(th, axis=(1, 2))
    t_int = jnp.sum(jnp.where(m, th, 0.0), axis=(1, 2))
    cnt = jnp.sum(jnp.where(m, 1.0, 0.0), axis=(1, 2))
    sig_sum = 0.5 * t_tot + 0.5 * jnp.float32(h * w)
    numer = t_int + cnt
    loss_pc = 1.0 - numer / (sig_sum + cnt + EPS_DICE)
    loss_ref[0] = (jnp.sum(loss_pc, keepdims=True) * (1.0 / c)).reshape(1, 1)


@jax.jit
def kernel(output, target):
    b, c, h, w = output.shape
    tgt = target.astype(jnp.int32)
    loss = pl.pallas_call(
        _dice_block_kernel,
        grid=(b,),
        in_specs=[
            pl.BlockSpec((1, c, h, w), lambda i: (i, 0, 0, 0)),
            pl.BlockSpec((1, h, w), lambda i: (i, 0, 0)),
        ],
        out_specs=pl.BlockSpec((1, 1, 1), lambda i: (i, 0, 0)),
        out_shape=jax.ShapeDtypeStruct((b, 1, 1), jnp.float32),
        compiler_params=pltpu.CompilerParams(
            dimension_semantics=("arbitrary",),
            vmem_limit_bytes=100 * 1024 * 1024,
        ),
    )(output, tgt)
    return loss[:, 0, 0]
